# Initial kernel scaffold; baseline (speedup 1.0000x reference)
import jax
import jax.numpy as jnp
from jax.experimental import pallas as pl


def kernel(x, edge_index, W1, b1, W2, b2, W3, b3):
    raise NotImplementedError("write your pallas kernel here")


# speedup vs baseline: 3.2803x; 3.2803x over previous
import functools

import jax
import jax.numpy as jnp
from jax import lax
from jax.experimental import pallas as pl
from jax.experimental.pallas import tpu as pltpu
from jax.experimental.pallas import tpu_sc as plsc

_N = 10000
_E = 320000
_W = 128
_NC = 2
_NS = 16
_NW = _NC * _NS
_CHUNK = 128
_EPT = 10112
_EPAD = _EPT * _NW
_NACC = 10240
_TROWS = _NACC // _NS
_BN = 80
_GRID = _N // _BN
_NB = _NACC // _BN

_mesh = plsc.VectorSubcoreMesh(core_axis_name="c", subcore_axis_name="s",
                               num_cores=_NC, num_subcores=_NS)


def _segsum_sc(table, src, dst, zeros, with_gather):
  n_chunks = _EPT // _CHUNK
  n_zfull = _TROWS // _CHUNK

  @functools.partial(
      pl.kernel,
      out_type=jax.ShapeDtypeStruct((_NC * _NACC, _W), jnp.float32),
      mesh=_mesh,
      scratch_types=[
          pltpu.VMEM((_CHUNK,), jnp.int32),
          pltpu.VMEM((_CHUNK,), jnp.int32),
          pltpu.VMEM((_CHUNK, _W), jnp.float32),
          pltpu.VMEM_SHARED((_NACC, _W), jnp.float32),
          pltpu.SemaphoreType.DMA,
      ],
  )
  def k(table_hbm, src_hbm, dst_hbm, zeros_hbm, out_hbm,
        src_v, dst_v, rows_v, acc, sem):
    c = lax.axis_index("c")
    s = lax.axis_index("s")
    wid = s * _NC + c
    pltpu.sync_copy(zeros_hbm, rows_v)
    row0 = s * _TROWS
    for j in range(n_zfull):
      pltpu.sync_copy(rows_v, acc.at[pl.ds(row0 + j * _CHUNK, _CHUNK)])
    plsc.subcore_barrier()
    if not with_gather:
      pltpu.sync_copy(table_hbm, rows_v)
    base = wid * _EPT

    def step(i, carry):
      off = base + i * _CHUNK
      pltpu.sync_copy(dst_hbm.at[pl.ds(off, _CHUNK)], dst_v)
      if with_gather:
        pltpu.sync_copy(src_hbm.at[pl.ds(off, _CHUNK)], src_v)
        pltpu.async_copy(table_hbm.at[src_v], rows_v, sem).wait()
      pltpu.sync_copy(rows_v, acc.at[dst_v], add=True)
      return carry

    lax.fori_loop(0, n_chunks, step, 0)
    plsc.subcore_barrier()
    pltpu.sync_copy(acc.at[pl.ds(row0, _TROWS)],
                    out_hbm.at[pl.ds(c * _NACC + row0, _TROWS)])

  return k(table, src, dst, zeros)


def _mm_xw1(x, w1):
  def body(x_ref, w_ref, o_ref):
    o_ref[...] = lax.dot_general(x_ref[...], w_ref[...],
                                 (((1,), (1,)), ((), ())),
                                 preferred_element_type=jnp.float32)

  return pl.pallas_call(
      body,
      grid=(_GRID,),
      in_specs=[
          pl.BlockSpec((_BN, 128), lambda i: (i, 0)),
          pl.BlockSpec((128, 128), lambda i: (0, 0)),
      ],
      out_specs=pl.BlockSpec((_BN, 128), lambda i: (i, 0)),
      out_shape=jax.ShapeDtypeStruct((_N, 128), jnp.float32),
  )(x, w1)


def _layer_tc(sf, degf, b, w):
  def body(p0_ref, p1_ref, d0_ref, d1_ref, b_ref, w_ref, o_ref, inv_ref):
    deg = d0_ref[:, 0:1] + d1_ref[:, 0:1]
    inv = 1.0 / jnp.maximum(deg, 1.0)
    ssum = p0_ref[...] + p1_ref[...]
    h = jnp.maximum(ssum * inv + b_ref[0:1, :], 0.0)
    o_ref[...] = lax.dot_general(h, w_ref[...], (((1,), (1,)), ((), ())),
                                 preferred_element_type=jnp.float32)
    inv_ref[...] = jnp.broadcast_to(inv, (_BN, 16))

  return pl.pallas_call(
      body,
      grid=(_GRID,),
      in_specs=[
          pl.BlockSpec((_BN, 128), lambda i: (i, 0)),
          pl.BlockSpec((_BN, 128), lambda i: (i + _NB, 0)),
          pl.BlockSpec((_BN, 128), lambda i: (i, 0)),
          pl.BlockSpec((_BN, 128), lambda i: (i + _NB, 0)),
          pl.BlockSpec((8, 128), lambda i: (0, 0)),
          pl.BlockSpec((128, 128), lambda i: (0, 0)),
      ],
      out_specs=[
          pl.BlockSpec((_BN, 128), lambda i: (i, 0)),
          pl.BlockSpec((_BN, 16), lambda i: (i, 0)),
      ],
      out_shape=[
          jax.ShapeDtypeStruct((_N, 128), jnp.float32),
          jax.ShapeDtypeStruct((_N, 16), jnp.float32),
      ],
  )(sf, sf, degf, degf, b, w)


def _layer3_tc(sf, inv, b, w3p):
  def body(p0_ref, p1_ref, inv_ref, b_ref, w_ref, o_ref):
    ssum = p0_ref[...] + p1_ref[...]
    h = jnp.maximum(ssum * inv_ref[:, 0:1] + b_ref[0:1, :], 0.0)
    o_ref[...] = lax.dot_general(h, w_ref[...], (((1,), (1,)), ((), ())),
                                 preferred_element_type=jnp.float32)

  return pl.pallas_call(
      body,
      grid=(_GRID,),
      in_specs=[
          pl.BlockSpec((_BN, 128), lambda i: (i, 0)),
          pl.BlockSpec((_BN, 128), lambda i: (i + _NB, 0)),
          pl.BlockSpec((_BN, 16), lambda i: (i, 0)),
          pl.BlockSpec((8, 128), lambda i: (0, 0)),
          pl.BlockSpec((128, 128), lambda i: (0, 0)),
      ],
      out_specs=pl.BlockSpec((_BN, 128), lambda i: (i, 0)),
      out_shape=jax.ShapeDtypeStruct((_N, 128), jnp.float32),
  )(sf, sf, inv, b, w3p)


def _final_tc(sf, inv, b3p):
  def body(p0_ref, p1_ref, inv_ref, b_ref, o_ref):
    ssum = p0_ref[...] + p1_ref[...]
    o_ref[...] = ssum * inv_ref[:, 0:1] + b_ref[0:1, :]

  return pl.pallas_call(
      body,
      grid=(_GRID,),
      in_specs=[
          pl.BlockSpec((_BN, 128), lambda i: (i, 0)),
          pl.BlockSpec((_BN, 128), lambda i: (i + _NB, 0)),
          pl.BlockSpec((_BN, 16), lambda i: (i, 0)),
          pl.BlockSpec((8, 128), lambda i: (0, 0)),
      ],
      out_specs=pl.BlockSpec((_BN, 128), lambda i: (i, 0)),
      out_shape=jax.ShapeDtypeStruct((_N, 128), jnp.float32),
  )(sf, sf, inv, b3p)


def kernel(x, edge_index, W1, b1, W2, b2, W3, b3):
  src = edge_index[0]
  dst = edge_index[1]
  npad = _EPAD - _E
  src_p = jnp.concatenate([src, jnp.zeros((npad,), jnp.int32)])
  dst_p = jnp.concatenate([dst, jnp.full((npad,), _N, jnp.int32)])

  b1b = jnp.broadcast_to(b1[None, :], (8, 128))
  b2b = jnp.broadcast_to(b2[None, :], (8, 128))
  b3p = jnp.concatenate([b3, jnp.zeros((88,), jnp.float32)])
  b3b = jnp.broadcast_to(b3p[None, :], (8, 128))
  w3p = jnp.concatenate([W3, jnp.zeros((88, 128), jnp.float32)], axis=0)

  zrows = jnp.zeros((_CHUNK, _W), jnp.float32)
  ones = jnp.ones((_CHUNK, _W), jnp.float32)

  degf = _segsum_sc(ones, src_p, dst_p, zrows, with_gather=False)
  p1 = _mm_xw1(x, W1)
  s1f = _segsum_sc(p1, src_p, dst_p, zrows, with_gather=True)
  p2, inv = _layer_tc(s1f, degf, b1b, W2)
  s2f = _segsum_sc(p2, src_p, dst_p, zrows, with_gather=True)
  p3 = _layer3_tc(s2f, inv, b2b, w3p)
  s3f = _segsum_sc(p3, src_p, dst_p, zrows, with_gather=True)
  out = _final_tc(s3f, inv, b3b)
  h = out[:, :40]

  total_comb_size = 3840000
  total_actv_size = 3840000
  return (h, total_comb_size, total_actv_size)
